# baseline (speedup 1.0000x reference)
import jax
import jax.numpy as jnp
from jax.experimental import pallas as pl
from jax.experimental.pallas import tpu as pltpu

_K = 8


def _moe_layer_kernel(x_ref, gate_w_ref, bias_ref, wgu_ref, wd_ref,
                      shgu_ref, shd_ref, out_ref, cw_ref):
    e = pl.program_id(0)
    x = x_ref[...]

    @pl.when(e == 0)
    def _router_and_shared():
        logits = jnp.dot(x, gate_w_ref[...], preferred_element_type=jnp.float32)
        scores = jax.nn.sigmoid(logits)
        sc = scores + bias_ref[...]
        T, E = sc.shape
        lane = jax.lax.broadcasted_iota(jnp.int32, (T, E), 1)
        selected = jnp.zeros((T, E), jnp.bool_)
        masked = sc
        for _ in range(_K):
            mx = jnp.max(masked, axis=1, keepdims=True)
            hit = masked == mx
            first = jnp.min(jnp.where(hit, lane, E), axis=1, keepdims=True)
            pick = lane == first
            selected = jnp.logical_or(selected, pick)
            masked = jnp.where(pick, -jnp.inf, masked)
        wsel = jnp.where(selected, scores, 0.0)
        wsum = jnp.sum(wsel, axis=1, keepdims=True)
        cw_ref[...] = wsel / (wsum + 1e-20)

        sgu = jnp.dot(x, shgu_ref[...], preferred_element_type=jnp.float32)
        sg, su = jnp.split(sgu, 2, axis=-1)
        out_ref[...] = jnp.dot(jax.nn.silu(sg) * su, shd_ref[...],
                               preferred_element_type=jnp.float32)

    xb = x.astype(jnp.bfloat16)
    eb = wgu_ref.shape[0]
    lane = jax.lax.broadcasted_iota(jnp.int32, cw_ref.shape, 1)
    acts = []
    for j in range(eb):
        gu = jnp.dot(xb, wgu_ref[j].astype(jnp.bfloat16),
                     preferred_element_type=jnp.float32)
        g, u = jnp.split(gu, 2, axis=-1)
        col = jnp.sum(jnp.where(lane == e * eb + j, cw_ref[...], 0.0),
                      axis=1, keepdims=True)
        acts.append(((jax.nn.silu(g) * u) * col).astype(jnp.bfloat16))
    act_cat = jnp.concatenate(acts, axis=-1)
    wd_flat = wd_ref[...].reshape(eb * wd_ref.shape[1], wd_ref.shape[2])
    out_ref[...] += jnp.dot(act_cat, wd_flat.astype(jnp.bfloat16),
                            preferred_element_type=jnp.float32)


def kernel(hidden_states, gate_w, expert_bias, w_gate_up, w_down,
           sh_gate_up, sh_down):
    orig_shape = hidden_states.shape
    x = hidden_states.reshape(-1, hidden_states.shape[-1])
    T, D = x.shape
    E = gate_w.shape[1]
    I2 = w_gate_up.shape[2]
    Dn = w_down.shape[2]
    S2 = sh_gate_up.shape[1]

    EB = 8
    out = pl.pallas_call(
        _moe_layer_kernel,
        grid=(E // EB,),
        in_specs=[
            pl.BlockSpec((T, D), lambda e: (0, 0)),
            pl.BlockSpec((D, E), lambda e: (0, 0)),
            pl.BlockSpec((1, E), lambda e: (0, 0)),
            pl.BlockSpec((EB, D, I2), lambda e: (e, 0, 0)),
            pl.BlockSpec((EB, w_down.shape[1], Dn), lambda e: (e, 0, 0)),
            pl.BlockSpec((D, S2), lambda e: (0, 0)),
            pl.BlockSpec((sh_down.shape[0], Dn), lambda e: (0, 0)),
        ],
        out_specs=pl.BlockSpec((T, Dn), lambda e: (0, 0)),
        out_shape=jax.ShapeDtypeStruct((T, Dn), jnp.float32),
        scratch_shapes=[pltpu.VMEM((T, E), jnp.float32)],
    )(x, gate_w, expert_bias.reshape(1, E), w_gate_up, w_down,
      sh_gate_up, sh_down)
    return out.reshape(orig_shape)


# speedup vs baseline: 1.0428x; 1.0428x over previous
import jax
import jax.numpy as jnp
from jax.experimental import pallas as pl
from jax.experimental.pallas import tpu as pltpu

_K = 8


def _moe_layer_kernel(x_ref, gate_w_ref, bias_ref, wgu_ref, wd_ref,
                      shgu_ref, shd_ref, out_ref, cw_ref, act_ref):
    e = pl.program_id(0)
    nsteps = pl.num_programs(0)
    x = x_ref[...]

    @pl.when(e == 0)
    def _router_and_shared():
        logits = jnp.dot(x, gate_w_ref[...], preferred_element_type=jnp.float32)
        scores = jax.nn.sigmoid(logits)
        sc = scores + bias_ref[...]
        T, E = sc.shape
        lane = jax.lax.broadcasted_iota(jnp.int32, (T, E), 1)
        selected = jnp.zeros((T, E), jnp.bool_)
        masked = sc
        for _ in range(_K):
            mx = jnp.max(masked, axis=1, keepdims=True)
            hit = masked == mx
            first = jnp.min(jnp.where(hit, lane, E), axis=1, keepdims=True)
            pick = lane == first
            selected = jnp.logical_or(selected, pick)
            masked = jnp.where(pick, -jnp.inf, masked)
        wsel = jnp.where(selected, scores, 0.0)
        wsum = jnp.sum(wsel, axis=1, keepdims=True)
        cw_ref[...] = wsel / (wsum + 1e-20)

        sgu = jnp.dot(x, shgu_ref[...], preferred_element_type=jnp.float32)
        sg, su = jnp.split(sgu, 2, axis=-1)
        out_ref[...] = jnp.dot(jax.nn.silu(sg) * su, shd_ref[...],
                               preferred_element_type=jnp.float32)

    @pl.when(e > 0)
    def _down_prev():
        wd_flat = wd_ref[...].reshape(-1, wd_ref.shape[2])
        out_ref[...] += jnp.dot(act_ref[...], wd_flat.astype(jnp.bfloat16),
                                preferred_element_type=jnp.float32)

    @pl.when(e < nsteps - 1)
    def _gu_cur():
        xb = x.astype(jnp.bfloat16)
        eb = wgu_ref.shape[0]
        lane = jax.lax.broadcasted_iota(jnp.int32, cw_ref.shape, 1)
        acts = []
        for j in range(eb):
            gu = jnp.dot(xb, wgu_ref[j].astype(jnp.bfloat16),
                         preferred_element_type=jnp.float32)
            g, u = jnp.split(gu, 2, axis=-1)
            col = jnp.sum(jnp.where(lane == e * eb + j, cw_ref[...], 0.0),
                          axis=1, keepdims=True)
            acts.append(((jax.nn.silu(g) * u) * col).astype(jnp.bfloat16))
        act_ref[...] = jnp.concatenate(acts, axis=-1)


def kernel(hidden_states, gate_w, expert_bias, w_gate_up, w_down,
           sh_gate_up, sh_down):
    orig_shape = hidden_states.shape
    x = hidden_states.reshape(-1, hidden_states.shape[-1])
    T, D = x.shape
    E = gate_w.shape[1]
    I2 = w_gate_up.shape[2]
    Ihid = w_down.shape[1]
    Dn = w_down.shape[2]
    S2 = sh_gate_up.shape[1]

    EB = 4
    NB = E // EB
    out = pl.pallas_call(
        _moe_layer_kernel,
        grid=(NB + 1,),
        in_specs=[
            pl.BlockSpec((T, D), lambda e: (0, 0)),
            pl.BlockSpec((D, E), lambda e: (0, 0)),
            pl.BlockSpec((1, E), lambda e: (0, 0)),
            pl.BlockSpec((EB, D, I2), lambda e: (jnp.minimum(e, NB - 1), 0, 0)),
            pl.BlockSpec((EB, Ihid, Dn), lambda e: (jnp.maximum(e - 1, 0), 0, 0)),
            pl.BlockSpec((D, S2), lambda e: (0, 0)),
            pl.BlockSpec((sh_down.shape[0], Dn), lambda e: (0, 0)),
        ],
        out_specs=pl.BlockSpec((T, Dn), lambda e: (0, 0)),
        out_shape=jax.ShapeDtypeStruct((T, Dn), jnp.float32),
        scratch_shapes=[pltpu.VMEM((T, E), jnp.float32),
                        pltpu.VMEM((T, EB * Ihid), jnp.bfloat16)],
    )(x, gate_w, expert_bias.reshape(1, E), w_gate_up, w_down,
      sh_gate_up, sh_down)
    return out.reshape(orig_shape)


# speedup vs baseline: 1.0527x; 1.0096x over previous
import jax
import jax.numpy as jnp
from jax.experimental import pallas as pl
from jax.experimental.pallas import tpu as pltpu

_K = 8


def _moe_layer_kernel(x_ref, gate_w_ref, bias_ref, wgu_ref, wd_ref,
                      shgu_ref, shd_ref, out_ref, cw_ref):
    e = pl.program_id(0)
    x = x_ref[...]

    @pl.when(e == 0)
    def _router_and_shared():
        logits = jnp.dot(x, gate_w_ref[...], preferred_element_type=jnp.float32)
        scores = jax.nn.sigmoid(logits)
        sc = scores + bias_ref[...]
        T, E = sc.shape
        lane = jax.lax.broadcasted_iota(jnp.int32, (T, E), 1)
        selected = jnp.zeros((T, E), jnp.bool_)
        masked = sc
        for _ in range(_K):
            mx = jnp.max(masked, axis=1, keepdims=True)
            hit = masked == mx
            first = jnp.min(jnp.where(hit, lane, E), axis=1, keepdims=True)
            pick = lane == first
            selected = jnp.logical_or(selected, pick)
            masked = jnp.where(pick, -jnp.inf, masked)
        wsel = jnp.where(selected, scores, 0.0)
        wsum = jnp.sum(wsel, axis=1, keepdims=True)
        cw_ref[...] = wsel / (wsum + 1e-20)

        sgu = jnp.dot(x, shgu_ref[...], preferred_element_type=jnp.float32)
        sg, su = jnp.split(sgu, 2, axis=-1)
        out_ref[...] = jnp.dot(jax.nn.silu(sg) * su, shd_ref[...],
                               preferred_element_type=jnp.float32)

    xb = x.astype(jnp.bfloat16)
    eb = wgu_ref.shape[0]
    lane = jax.lax.broadcasted_iota(jnp.int32, cw_ref.shape, 1)
    acts = []
    for j in range(eb):
        gu = jnp.dot(xb, wgu_ref[j].astype(jnp.bfloat16),
                     preferred_element_type=jnp.float32)
        g, u = jnp.split(gu, 2, axis=-1)
        col = jnp.sum(jnp.where(lane == e * eb + j, cw_ref[...], 0.0),
                      axis=1, keepdims=True)
        acts.append(((jax.nn.silu(g) * u) * col).astype(jnp.bfloat16))
    act_cat = jnp.concatenate(acts, axis=-1)
    wd_flat = wd_ref[...].reshape(eb * wd_ref.shape[1], wd_ref.shape[2])
    out_ref[...] += jnp.dot(act_cat, wd_flat.astype(jnp.bfloat16),
                            preferred_element_type=jnp.float32)


def kernel(hidden_states, gate_w, expert_bias, w_gate_up, w_down,
           sh_gate_up, sh_down):
    orig_shape = hidden_states.shape
    x = hidden_states.reshape(-1, hidden_states.shape[-1])
    T, D = x.shape
    E = gate_w.shape[1]
    I2 = w_gate_up.shape[2]
    Dn = w_down.shape[2]
    S2 = sh_gate_up.shape[1]

    EB = 4
    out = pl.pallas_call(
        _moe_layer_kernel,
        grid=(E // EB,),
        in_specs=[
            pl.BlockSpec((T, D), lambda e: (0, 0)),
            pl.BlockSpec((D, E), lambda e: (0, 0)),
            pl.BlockSpec((1, E), lambda e: (0, 0)),
            pl.BlockSpec((EB, D, I2), lambda e: (e, 0, 0)),
            pl.BlockSpec((EB, w_down.shape[1], Dn), lambda e: (e, 0, 0)),
            pl.BlockSpec((D, S2), lambda e: (0, 0)),
            pl.BlockSpec((sh_down.shape[0], Dn), lambda e: (0, 0)),
        ],
        out_specs=pl.BlockSpec((T, Dn), lambda e: (0, 0)),
        out_shape=jax.ShapeDtypeStruct((T, Dn), jnp.float32),
        scratch_shapes=[pltpu.VMEM((T, E), jnp.float32)],
    )(x, gate_w, expert_bias.reshape(1, E), w_gate_up, w_down,
      sh_gate_up, sh_down)
    return out.reshape(orig_shape)
